# baseline (speedup 1.0000x reference)
import functools

import jax
import jax.numpy as jnp
from jax import lax
from jax.experimental import pallas as pl
from jax.experimental.pallas import tpu as pltpu
from jax.experimental.pallas import tpu_sc as plsc

B = 16384
E = 16
NC = 2
NS = 16
NW = NC * NS
BPW = B // NW
CH = 128
NCH = BPW // CH


def _sc_gather_dot(u_idx2d, r_idx2d, u_emb, r_emb, u_bias, r_bias):
    mesh = plsc.VectorSubcoreMesh(core_axis_name="c", subcore_axis_name="s")

    @functools.partial(
        pl.kernel,
        mesh=mesh,
        out_type=(
            jax.ShapeDtypeStruct((NW * E,), jnp.float32),
            jax.ShapeDtypeStruct((B,), jnp.float32),
        ),
        scratch_types=[
            pltpu.VMEM((NCH, CH), jnp.int32),
            pltpu.VMEM((NCH, CH), jnp.int32),
            pltpu.VMEM((BPW, E), jnp.float32),
            pltpu.VMEM((BPW, E), jnp.float32),
            pltpu.VMEM((BPW,), jnp.float32),
            pltpu.VMEM((BPW,), jnp.float32),
            pltpu.VMEM((BPW,), jnp.float32),
            pltpu.VMEM((E,), jnp.float32),
            pltpu.SemaphoreType.DMA,
        ],
        compiler_params=pltpu.CompilerParams(use_tc_tiling_on_sc=False),
    )
    def k(u_idx_hbm, r_idx_hbm, u_emb_hbm, r_emb_hbm, u_bias_hbm, r_bias_hbm,
          partial_hbm, ubrb_hbm, idx_u, idx_r, u_rows, r_rows, ub_v, rb_v,
          ubrb_v, acc_v, sem):
        wid = lax.axis_index("s") * NC + lax.axis_index("c")
        base = pl.multiple_of(wid * BPW, 8)
        row0 = wid * NCH

        pltpu.sync_copy(u_idx_hbm.at[pl.ds(row0, NCH)], idx_u)
        pltpu.sync_copy(r_idx_hbm.at[pl.ds(row0, NCH)], idx_r)

        copies = []
        for j in range(NCH):
            sl = pl.ds(j * CH, CH)
            copies.append(pltpu.async_copy(u_emb_hbm.at[idx_u.at[j]], u_rows.at[sl], sem))
            copies.append(pltpu.async_copy(r_emb_hbm.at[idx_r.at[j]], r_rows.at[sl], sem))
            copies.append(pltpu.async_copy(u_bias_hbm.at[idx_u.at[j]], ub_v.at[sl], sem))
            copies.append(pltpu.async_copy(r_bias_hbm.at[idx_r.at[j]], rb_v.at[sl], sem))
        for c in copies:
            c.wait()

        def dot_body(i, acc):
            return acc + u_rows[i, :] * r_rows[i, :]

        acc_v[...] = lax.fori_loop(0, BPW, dot_body, jnp.zeros((E,), jnp.float32))
        pltpu.sync_copy(acc_v, partial_hbm.at[pl.ds(pl.multiple_of(wid * E, 8), E)])

        for i in range(BPW // E):
            sl = pl.ds(i * E, E)
            ubrb_v[sl] = ub_v[sl] + rb_v[sl]
        pltpu.sync_copy(ubrb_v, ubrb_hbm.at[pl.ds(base, BPW)])

    return k(u_idx2d, r_idx2d, u_emb, r_emb, u_bias, r_bias)


def _tc_finish(partials_2d, ubrb_2d):
    def body(p_ref, x_ref, o_ref):
        s = jnp.sum(p_ref[...])
        o_ref[...] = jax.nn.sigmoid(x_ref[...] + s)

    return pl.pallas_call(
        body,
        out_shape=jax.ShapeDtypeStruct(ubrb_2d.shape, jnp.float32),
    )(partials_2d, ubrb_2d)


def kernel(inputs, user_embedding, user_bias, resto_embedding, resto_bias):
    idx = inputs.astype(jnp.int32)
    u_idx2d = idx[:, 0].reshape(B // CH, CH)
    r_idx2d = idx[:, 1].reshape(B // CH, CH)
    partials, ubrb = _sc_gather_dot(
        u_idx2d, r_idx2d, user_embedding, resto_embedding,
        user_bias.reshape(-1), resto_bias.reshape(-1))
    out = _tc_finish(partials.reshape(NW * E // 128, 128), ubrb.reshape(B // 128, 128))
    return out.reshape(B, 1)
